# baseline (speedup 1.0000x reference)
import functools

import jax
import jax.numpy as jnp
from jax.experimental import pallas as pl
from jax.experimental.pallas import tpu as pltpu
from jax.experimental.pallas import tpu_sc as plsc

VOCAB = 1000
EMB = 128
FIELDS = 26
SUB = 8

G = 2


def kernel(indices, tables):
    batch = indices.shape[0]
    ngrp = batch // SUB
    win = G * SUB * FIELDS
    n = batch * FIELDS
    flat_tables = tables.reshape(FIELDS * VOCAB, EMB)
    offs = (jnp.arange(FIELDS, dtype=indices.dtype) * VOCAB)[None, :]
    p = jnp.arange(n, dtype=jnp.int32)
    grp = SUB * FIELDS
    perm = grp * (p // grp) + FIELDS * (p % SUB) + (p % grp) // SUB
    pidx = jnp.take((indices + offs).reshape(n), perm)

    mesh = plsc.VectorSubcoreMesh(core_axis_name="core", subcore_axis_name="subcore")

    @functools.partial(
        pl.kernel,
        out_type=jax.ShapeDtypeStruct((ngrp, FIELDS, SUB, EMB), tables.dtype),
        mesh=mesh,
        scratch_types=[pltpu.VMEM((win,), jnp.int32)],
    )
    def gather_kernel(x_hbm, i_hbm, o_hbm, idx_v):
        def body(grid_idx, o_vmem):
            (i,) = grid_idx
            pltpu.sync_copy(i_hbm.at[pl.ds(i * win, win)], idx_v)
            pltpu.sync_copy(x_hbm.at[idx_v], o_vmem.reshape(win, EMB))

        pltpu.emit_pipeline(
            body,
            grid=(ngrp // G,),
            out_specs=[
                pl.BlockSpec((G, FIELDS, SUB, EMB), index_map=lambda i: (i, 0, 0, 0))
            ],
            core_axis_name=("core", "subcore"),
            dimension_semantics=(pltpu.PARALLEL,),
            _explicit_indices=True,
        )(o_hbm)

    out4 = gather_kernel(flat_tables, pidx)
    return out4.transpose(0, 2, 1, 3).reshape(batch, FIELDS * EMB)


# speedup vs baseline: 1.2885x; 1.2885x over previous
import dataclasses
import functools

import jax
import jax.numpy as jnp
from jax import lax
from jax.experimental import pallas as pl
from jax.experimental.pallas import tpu as pltpu
from jax.experimental.pallas import tpu_sc as plsc

VOCAB = 1000
EMB = 128
FIELDS = 26
SUB = 8
NW = 32

WIN = SUB * FIELDS


def kernel(indices, tables):
    batch = indices.shape[0]
    ngrp = batch // SUB
    rows_w = batch // NW
    grp_w = ngrp // NW
    n = batch * FIELDS
    flat_tables = tables.reshape(FIELDS * VOCAB, EMB)
    offs = (jnp.arange(FIELDS, dtype=indices.dtype) * VOCAB)[None, :]
    biased = (indices + offs).reshape(n)

    mesh = plsc.VectorSubcoreMesh(core_axis_name="core", subcore_axis_name="subcore")

    cp = pltpu.CompilerParams()
    if "needs_layout_passes" in pltpu.CompilerParams.__dataclass_fields__:
        cp = dataclasses.replace(cp, needs_layout_passes=False)

    @functools.partial(
        pl.kernel,
        out_type=jax.ShapeDtypeStruct((ngrp, FIELDS, SUB, EMB), tables.dtype),
        mesh=mesh,
        scratch_types=[
            pltpu.VMEM((rows_w * FIELDS,), jnp.int32),
            pltpu.VMEM((rows_w * FIELDS,), jnp.int32),
        ],
        compiler_params=cp,
    )
    def gather_kernel(x_hbm, i_hbm, o_hbm, raw_v, idx_v):
        cid = lax.axis_index(("core", "subcore"))

        nw_ = rows_w * FIELDS
        pltpu.sync_copy(i_hbm.at[pl.ds(cid * nw_, nw_)], raw_v)

        lane = lax.broadcasted_iota(jnp.int32, (16,), 0)
        bi16 = lane % SUB
        fh16 = lane // SUB

        @pl.loop(0, grp_w)
        def _(g):
            @pl.loop(0, FIELDS // 2)
            def _(j):
                src16 = g * WIN + FIELDS * bi16 + 2 * j + fh16
                vals = plsc.load_gather(raw_v, [src16])
                idx_v[pl.ds(g * WIN + j * 16, 16)] = vals

        def body(grid_idx, o_vmem):
            (i,) = grid_idx
            li = i - cid * grp_w
            pltpu.sync_copy(
                x_hbm.at[idx_v.at[pl.ds(li * WIN, WIN)]],
                o_vmem.reshape(WIN, EMB),
            )

        pltpu.emit_pipeline(
            body,
            grid=(ngrp,),
            out_specs=[
                pl.BlockSpec((1, FIELDS, SUB, EMB), index_map=lambda i: (i, 0, 0, 0))
            ],
            core_axis_name=("core", "subcore"),
            dimension_semantics=(pltpu.PARALLEL,),
            _explicit_indices=True,
        )(o_hbm)

    out4 = gather_kernel(flat_tables, biased)
    return out4.transpose(0, 2, 1, 3).reshape(batch, FIELDS * EMB)


# speedup vs baseline: 1.3170x; 1.0222x over previous
import dataclasses
import functools

import jax
import jax.numpy as jnp
from jax import lax
from jax.experimental import pallas as pl
from jax.experimental.pallas import tpu as pltpu
from jax.experimental.pallas import tpu_sc as plsc

VOCAB = 1000
EMB = 128
FIELDS = 26
SUB = 8
NW = 32

G = 2
WIN = SUB * FIELDS
NHALF = 2


def kernel(indices, tables):
    batch = indices.shape[0]
    ngrp = batch // SUB
    grp_w = ngrp // NW
    n = batch * FIELDS
    nw_ = n // NW
    nh = nw_ // NHALF
    gh = grp_w // NHALF
    flat_tables = tables.reshape(FIELDS * VOCAB, EMB)
    offs = (jnp.arange(FIELDS, dtype=indices.dtype) * VOCAB)[None, :]
    biased = (indices + offs).reshape(n)

    mesh = plsc.VectorSubcoreMesh(core_axis_name="core", subcore_axis_name="subcore")

    cp = pltpu.CompilerParams()
    if "needs_layout_passes" in pltpu.CompilerParams.__dataclass_fields__:
        cp = dataclasses.replace(cp, needs_layout_passes=False)

    @functools.partial(
        pl.kernel,
        out_type=jax.ShapeDtypeStruct((ngrp, FIELDS, SUB, EMB), tables.dtype),
        mesh=mesh,
        scratch_types=[
            pltpu.VMEM((nh,), jnp.int32),
            pltpu.VMEM((nw_,), jnp.int32),
        ],
        compiler_params=cp,
    )
    def gather_kernel(x_hbm, i_hbm, o_hbm, raw_v, idx_v):
        cid = lax.axis_index(("core", "subcore"))

        lane = lax.broadcasted_iota(jnp.int32, (16,), 0)
        bi16 = lane % SUB
        fh16 = lane // SUB

        @pl.loop(0, NHALF)
        def _(h):
            pltpu.sync_copy(i_hbm.at[pl.ds(cid * nw_ + h * nh, nh)], raw_v)

            @pl.loop(0, gh)
            def _(g):
                @pl.loop(0, FIELDS // 2)
                def _(j):
                    src16 = g * WIN + FIELDS * bi16 + 2 * j + fh16
                    vals = plsc.load_gather(raw_v, [src16])
                    idx_v[pl.ds((h * gh + g) * WIN + j * 16, 16)] = vals

        def body(grid_idx, o_vmem):
            (i,) = grid_idx
            li = i - cid * (grp_w // G)
            pltpu.sync_copy(
                x_hbm.at[idx_v.at[pl.ds(li * G * WIN, G * WIN)]],
                o_vmem.reshape(G * WIN, EMB),
            )

        pltpu.emit_pipeline(
            body,
            grid=(ngrp // G,),
            out_specs=[
                pl.BlockSpec((G, FIELDS, SUB, EMB), index_map=lambda i: (i, 0, 0, 0))
            ],
            core_axis_name=("core", "subcore"),
            dimension_semantics=(pltpu.PARALLEL,),
            _explicit_indices=True,
        )(o_hbm)

    out4 = gather_kernel(flat_tables, biased)
    return out4.transpose(0, 2, 1, 3).reshape(batch, FIELDS * EMB)


# speedup vs baseline: 1.3423x; 1.0192x over previous
import dataclasses
import functools

import jax
import jax.numpy as jnp
from jax import lax
from jax.experimental import pallas as pl
from jax.experimental.pallas import tpu as pltpu
from jax.experimental.pallas import tpu_sc as plsc

VOCAB = 1000
EMB = 128
FIELDS = 26
SUB = 8
NW = 32

G = 2
WIN = SUB * FIELDS
NHALF = 2


def kernel(indices, tables):
    batch = indices.shape[0]
    ngrp = batch // SUB
    grp_w = ngrp // NW
    n = batch * FIELDS
    nw_ = n // NW
    nh = nw_ // NHALF
    gh = grp_w // NHALF
    flat_tables = tables.reshape(FIELDS * VOCAB, EMB)
    offs = (jnp.arange(FIELDS, dtype=indices.dtype) * VOCAB)[None, :]
    biased = (indices + offs).reshape(n)

    mesh = plsc.VectorSubcoreMesh(core_axis_name="core", subcore_axis_name="subcore")

    cp = pltpu.CompilerParams()
    if "needs_layout_passes" in pltpu.CompilerParams.__dataclass_fields__:
        cp = dataclasses.replace(cp, needs_layout_passes=False)

    @functools.partial(
        pl.kernel,
        out_type=jax.ShapeDtypeStruct((ngrp, FIELDS, SUB, EMB), tables.dtype),
        mesh=mesh,
        scratch_types=[
            pltpu.VMEM((nh,), jnp.int32),
            pltpu.VMEM((nw_,), jnp.int32),
            pltpu.SemaphoreType.DMA,
            pltpu.SemaphoreType.DMA,
        ],
        compiler_params=cp,
    )
    def gather_kernel(x_hbm, i_hbm, o_hbm, raw_v, idx_v, sem_a, sem_b):
        cid = lax.axis_index(("core", "subcore"))

        lane = lax.broadcasted_iota(jnp.int32, (16,), 0)
        bi16 = lane % SUB
        fh16 = lane // SUB

        @pl.loop(0, NHALF)
        def _(h):
            pltpu.sync_copy(i_hbm.at[pl.ds(cid * nw_ + h * nh, nh)], raw_v)

            @pl.loop(0, gh)
            def _(g):
                @pl.loop(0, FIELDS // 2)
                def _(j):
                    src16 = g * WIN + FIELDS * bi16 + 2 * j + fh16
                    vals = plsc.load_gather(raw_v, [src16])
                    idx_v[pl.ds((h * gh + g) * WIN + j * 16, 16)] = vals

        def body(grid_idx, o_vmem):
            (i,) = grid_idx
            li = i - cid * (grp_w // G)
            o_flat = o_vmem.reshape(G * WIN, EMB)
            ca = pltpu.make_async_copy(
                x_hbm.at[idx_v.at[pl.ds(li * G * WIN, WIN)]],
                o_flat.at[pl.ds(0, WIN), :],
                sem_a,
            )
            cb = pltpu.make_async_copy(
                x_hbm.at[idx_v.at[pl.ds(li * G * WIN + WIN, WIN)]],
                o_flat.at[pl.ds(WIN, WIN), :],
                sem_b,
            )
            ca.start()
            cb.start()
            ca.wait()
            cb.wait()

        pltpu.emit_pipeline(
            body,
            grid=(ngrp // G,),
            out_specs=[
                pl.BlockSpec((G, FIELDS, SUB, EMB), index_map=lambda i: (i, 0, 0, 0))
            ],
            core_axis_name=("core", "subcore"),
            dimension_semantics=(pltpu.PARALLEL,),
            _explicit_indices=True,
        )(o_hbm)

    out4 = gather_kernel(flat_tables, biased)
    return out4.transpose(0, 2, 1, 3).reshape(batch, FIELDS * EMB)
